# baseline (speedup 1.0000x reference)
import functools

import jax
import jax.numpy as jnp
from jax import lax
from jax.experimental import pallas as pl
from jax.experimental.pallas import tpu as pltpu, tpu_sc as plsc

_INFO = plsc.get_sparse_core_info()
_NC = _INFO.num_cores
_NS = _INFO.num_subcores
_NW = _NC * _NS

_CH = 50
_NBUF = 4
_WIN = 40
_ZR = 16


def _row_split(N):
    n0 = (N // _NS) // 8 * 8
    tail = N - n0 * _NS
    assert n0 % _ZR == 0 and tail % 8 == 0 and tail <= _ZR
    return n0, tail


def _make_deg_kernel(N, E):
    rows_per_tile = (E // _CH) // _NW
    n0, tail = _row_split(N)
    mesh = plsc.VectorSubcoreMesh(core_axis_name="c", subcore_axis_name="s")

    @functools.partial(
        pl.kernel,
        out_type=[
            jax.ShapeDtypeStruct((N, 16), jnp.float32),
            jax.ShapeDtypeStruct((N, 16), jnp.float32),
        ],
        mesh=mesh,
        scratch_types=[
            pltpu.VMEM((rows_per_tile, _CH), jnp.int32),
            pltpu.VMEM((_CH, 16), jnp.float32),
            pltpu.VMEM((_ZR, 16), jnp.float32),
            pltpu.VMEM_SHARED((N, 16), jnp.float32),
        ],
    )
    def deg_kernel(e3_hbm, out0, out1, dst_v, ones_v, zero_v, deg_sh):
        cid = lax.axis_index("c")
        sid = lax.axis_index("s")
        wid = cid * _NS + sid

        one16 = jnp.ones((16,), jnp.float32)
        zero16 = jnp.zeros((16,), jnp.float32)
        for r in range(_CH):
            ones_v[r] = one16
        for r in range(_ZR):
            zero_v[r] = zero16

        pltpu.sync_copy(
            e3_hbm.at[1, pl.ds(wid * rows_per_tile, rows_per_tile)], dst_v)

        def _zero(k, _):
            pltpu.sync_copy(zero_v, deg_sh.at[pl.ds(sid * n0 + k * _ZR, _ZR)])
            return 0

        lax.fori_loop(0, n0 // _ZR, _zero, 0)

        @pl.when(sid == _NS - 1)
        def _():
            pltpu.sync_copy(zero_v.at[pl.ds(0, tail)],
                            deg_sh.at[pl.ds(_NS * n0, tail)])

        plsc.subcore_barrier()

        def _hist(c, _):
            pltpu.sync_copy(ones_v, deg_sh.at[dst_v.at[c]], add=True)
            return 0

        lax.fori_loop(0, rows_per_tile, _hist, 0)
        plsc.subcore_barrier()

        def _dump(out):
            sl = pl.ds(sid * n0, n0)
            pltpu.sync_copy(deg_sh.at[sl], out.at[sl])

            @pl.when(sid == _NS - 1)
            def _():
                tl = pl.ds(_NS * n0, tail)
                pltpu.sync_copy(deg_sh.at[tl], out.at[tl])

        @pl.when(cid == 0)
        def _():
            _dump(out0)

        @pl.when(cid == 1)
        def _():
            _dump(out1)

    return deg_kernel


def _make_edge_kernel(N, E, D):
    rows_per_tile = (E // _CH) // _NW
    n0, tail = _row_split(N)
    assert rows_per_tile % _WIN == 0 and _WIN % _NBUF == 0 and _WIN % 8 == 0
    mesh = plsc.VectorSubcoreMesh(core_axis_name="c", subcore_axis_name="s")

    @functools.partial(
        pl.kernel,
        out_type=[
            jax.ShapeDtypeStruct((N, D), jnp.float32),
            jax.ShapeDtypeStruct((N, D), jnp.float32),
        ],
        mesh=mesh,
        scratch_types=[
            pltpu.VMEM((_WIN, _CH), jnp.int32),
            pltpu.VMEM((_WIN, _CH), jnp.int32),
        ] + [pltpu.VMEM((_CH, D), jnp.float32)] * _NBUF
          + [
            pltpu.VMEM((_ZR, D), jnp.float32),
            pltpu.VMEM_SHARED((N, D), jnp.float32),
        ] + [pltpu.SemaphoreType.DMA] * _NBUF,
    )
    def edge_kernel(e3_hbm, g_hbm, out0, out1,
                    src_v, dst_v, *rest):
        bufs = rest[:_NBUF]
        zero_v = rest[_NBUF]
        acc_sh = rest[_NBUF + 1]
        sems = rest[_NBUF + 2:]
        cid = lax.axis_index("c")
        sid = lax.axis_index("s")
        wid = cid * _NS + sid

        zero16 = jnp.zeros((16,), jnp.float32)
        for r in range(_ZR):
            for l in range(D // 16):
                zero_v[r, pl.ds(l * 16, 16)] = zero16

        def _zero(k, _):
            pltpu.sync_copy(zero_v,
                            acc_sh.at[pl.ds(sid * n0 + k * _ZR, _ZR)])
            return 0

        lax.fori_loop(0, n0 // _ZR, _zero, 0)

        @pl.when(sid == _NS - 1)
        def _():
            pltpu.sync_copy(zero_v.at[pl.ds(0, tail)],
                            acc_sh.at[pl.ds(_NS * n0, tail)])

        plsc.subcore_barrier()

        def _gather(c, buf, s):
            pltpu.async_copy(g_hbm.at[src_v.at[c]], buf, s)

        def _wait(buf, s):
            pltpu.make_async_copy(g_hbm.at[src_v.at[0]], buf, s).wait()

        def _edges(i, _):
            c0 = _NBUF * i
            _gather(c0 + _NBUF - 1, bufs[-1], sems[-1])
            for k in range(_NBUF):
                _wait(bufs[k], sems[k])
                pltpu.sync_copy(bufs[k], acc_sh.at[dst_v.at[c0 + k]],
                                add=True)

                @pl.when(c0 + _NBUF + k < _WIN)
                def _():
                    _gather(c0 + _NBUF + k, bufs[k], sems[k])

            return 0

        def _window(w, _):
            base = wid * rows_per_tile + w * _WIN
            pltpu.sync_copy(e3_hbm.at[0, pl.ds(base, _WIN)], src_v)
            pltpu.sync_copy(e3_hbm.at[1, pl.ds(base, _WIN)], dst_v)
            for k in range(_NBUF - 1):
                _gather(k, bufs[k], sems[k])
            lax.fori_loop(0, _WIN // _NBUF, _edges, 0)
            return 0

        lax.fori_loop(0, rows_per_tile // _WIN, _window, 0)
        plsc.subcore_barrier()

        def _dump(out):
            sl = pl.ds(sid * n0, n0)
            pltpu.sync_copy(acc_sh.at[sl], out.at[sl])

            @pl.when(sid == _NS - 1)
            def _():
                tl = pl.ds(_NS * n0, tail)
                pltpu.sync_copy(acc_sh.at[tl], out.at[tl])

        @pl.when(cid == 0)
        def _():
            _dump(out0)

        @pl.when(cid == 1)
        def _():
            _dump(out1)

    return edge_kernel


def _matmul_tc(x_ref, w_ref, h_ref):
    h_ref[...] = jnp.dot(x_ref[...], w_ref[...],
                         preferred_element_type=jnp.float32)


def _dinv_scale_tc(p0_ref, p1_ref, h_ref, dinv_ref, g_ref):
    deg = 1.0 + p0_ref[:, 0:1] + p1_ref[:, 0:1]
    dinv = lax.rsqrt(deg)
    dinv_ref[...] = dinv
    g_ref[...] = h_ref[...] * dinv


def _combine_tc(g_ref, p0_ref, p1_ref, dinv_ref, b_ref, out_ref):
    s = g_ref[...] + p0_ref[...] + p1_ref[...]
    out_ref[...] = dinv_ref[...] * s + b_ref[...]


def kernel(x, edge_index, W, b):
    N, D_in = x.shape
    D_out = W.shape[1]
    E = edge_index.shape[1]

    e3 = edge_index.reshape(2, E // _CH, _CH)

    nb = 2000
    grid = (N // nb,)

    h = pl.pallas_call(
        _matmul_tc,
        grid=grid,
        in_specs=[
            pl.BlockSpec((nb, D_in), lambda i: (i, 0)),
            pl.BlockSpec((D_in, D_out), lambda i: (0, 0)),
        ],
        out_specs=pl.BlockSpec((nb, D_out), lambda i: (i, 0)),
        out_shape=jax.ShapeDtypeStruct((N, D_out), jnp.float32),
    )(x, W)

    hp0, hp1 = _make_deg_kernel(N, E)(e3)

    dinv, g = pl.pallas_call(
        _dinv_scale_tc,
        grid=grid,
        in_specs=[
            pl.BlockSpec((nb, 16), lambda i: (i, 0)),
            pl.BlockSpec((nb, 16), lambda i: (i, 0)),
            pl.BlockSpec((nb, D_out), lambda i: (i, 0)),
        ],
        out_specs=[
            pl.BlockSpec((nb, 1), lambda i: (i, 0)),
            pl.BlockSpec((nb, D_out), lambda i: (i, 0)),
        ],
        out_shape=[
            jax.ShapeDtypeStruct((N, 1), jnp.float32),
            jax.ShapeDtypeStruct((N, D_out), jnp.float32),
        ],
    )(hp0, hp1, h)

    p0, p1 = _make_edge_kernel(N, E, D_out)(e3, g)

    out = pl.pallas_call(
        _combine_tc,
        grid=grid,
        in_specs=[
            pl.BlockSpec((nb, D_out), lambda i: (i, 0)),
            pl.BlockSpec((nb, D_out), lambda i: (i, 0)),
            pl.BlockSpec((nb, D_out), lambda i: (i, 0)),
            pl.BlockSpec((nb, 1), lambda i: (i, 0)),
            pl.BlockSpec((1, D_out), lambda i: (0, 0)),
        ],
        out_specs=pl.BlockSpec((nb, D_out), lambda i: (i, 0)),
        out_shape=jax.ShapeDtypeStruct((N, D_out), jnp.float32),
    )(g, p0, p1, dinv, b.reshape(1, D_out))

    return out


# speedup vs baseline: 1.1215x; 1.1215x over previous
import functools

import jax
import jax.numpy as jnp
from jax import lax
from jax.experimental import pallas as pl
from jax.experimental.pallas import tpu as pltpu, tpu_sc as plsc

_INFO = plsc.get_sparse_core_info()
_NC = _INFO.num_cores
_NS = _INFO.num_subcores
_NW = _NC * _NS

_CH = 125
_NBUF = 2
_WIN = 40
_ZR = 16


def _row_split(N):
    n0 = (N // _NS) // 8 * 8
    tail = N - n0 * _NS
    assert n0 % _ZR == 0 and tail % 8 == 0 and tail <= _ZR
    return n0, tail


def _make_deg_kernel(N, E):
    rows_per_tile = (E // _CH) // _NW
    n0, tail = _row_split(N)
    mesh = plsc.VectorSubcoreMesh(core_axis_name="c", subcore_axis_name="s")

    @functools.partial(
        pl.kernel,
        out_type=[
            jax.ShapeDtypeStruct((N, 16), jnp.float32),
            jax.ShapeDtypeStruct((N, 16), jnp.float32),
        ],
        mesh=mesh,
        scratch_types=[
            pltpu.VMEM((rows_per_tile, _CH), jnp.int32),
            pltpu.VMEM((_CH, 16), jnp.float32),
            pltpu.VMEM((_ZR, 16), jnp.float32),
            pltpu.VMEM_SHARED((N, 16), jnp.float32),
        ],
    )
    def deg_kernel(e3_hbm, out0, out1, dst_v, ones_v, zero_v, deg_sh):
        cid = lax.axis_index("c")
        sid = lax.axis_index("s")
        wid = cid * _NS + sid

        one16 = jnp.ones((16,), jnp.float32)
        zero16 = jnp.zeros((16,), jnp.float32)
        for r in range(_CH):
            ones_v[r] = one16
        for r in range(_ZR):
            zero_v[r] = zero16

        pltpu.sync_copy(e3_hbm.at[1, wid], dst_v)

        def _zero(k, _):
            pltpu.sync_copy(zero_v, deg_sh.at[pl.ds(sid * n0 + k * _ZR, _ZR)])
            return 0

        lax.fori_loop(0, n0 // _ZR, _zero, 0)

        @pl.when(sid == _NS - 1)
        def _():
            pltpu.sync_copy(zero_v.at[pl.ds(0, tail)],
                            deg_sh.at[pl.ds(_NS * n0, tail)])

        plsc.subcore_barrier()

        def _hist(c, _):
            pltpu.sync_copy(ones_v, deg_sh.at[dst_v.at[c]], add=True)
            return 0

        lax.fori_loop(0, rows_per_tile, _hist, 0)
        plsc.subcore_barrier()

        def _dump(out):
            sl = pl.ds(sid * n0, n0)
            pltpu.sync_copy(deg_sh.at[sl], out.at[sl])

            @pl.when(sid == _NS - 1)
            def _():
                tl = pl.ds(_NS * n0, tail)
                pltpu.sync_copy(deg_sh.at[tl], out.at[tl])

        @pl.when(cid == 0)
        def _():
            _dump(out0)

        @pl.when(cid == 1)
        def _():
            _dump(out1)

    return deg_kernel


def _make_edge_kernel(N, E, D):
    rows_per_tile = (E // _CH) // _NW
    n0, tail = _row_split(N)
    assert rows_per_tile % _WIN == 0 and _WIN % _NBUF == 0
    mesh = plsc.VectorSubcoreMesh(core_axis_name="c", subcore_axis_name="s")

    @functools.partial(
        pl.kernel,
        out_type=[
            jax.ShapeDtypeStruct((N, D), jnp.float32),
            jax.ShapeDtypeStruct((N, D), jnp.float32),
        ],
        mesh=mesh,
        scratch_types=[
            pltpu.VMEM((_WIN, _CH), jnp.int32),
            pltpu.VMEM((_WIN, _CH), jnp.int32),
        ] + [pltpu.VMEM((_CH, D), jnp.float32)] * _NBUF
          + [
            pltpu.VMEM((_ZR, D), jnp.float32),
            pltpu.VMEM_SHARED((N, D), jnp.float32),
        ] + [pltpu.SemaphoreType.DMA] * _NBUF,
    )
    def edge_kernel(e3_hbm, g_hbm, out0, out1,
                    src_v, dst_v, *rest):
        bufs = rest[:_NBUF]
        zero_v = rest[_NBUF]
        acc_sh = rest[_NBUF + 1]
        sems = rest[_NBUF + 2:]
        cid = lax.axis_index("c")
        sid = lax.axis_index("s")
        wid = cid * _NS + sid

        zero16 = jnp.zeros((16,), jnp.float32)
        for r in range(_ZR):
            for l in range(D // 16):
                zero_v[r, pl.ds(l * 16, 16)] = zero16

        def _zero(k, _):
            pltpu.sync_copy(zero_v,
                            acc_sh.at[pl.ds(sid * n0 + k * _ZR, _ZR)])
            return 0

        lax.fori_loop(0, n0 // _ZR, _zero, 0)

        @pl.when(sid == _NS - 1)
        def _():
            pltpu.sync_copy(zero_v.at[pl.ds(0, tail)],
                            acc_sh.at[pl.ds(_NS * n0, tail)])

        plsc.subcore_barrier()

        def _gather(c, buf, s):
            pltpu.async_copy(g_hbm.at[src_v.at[c]], buf, s)

        def _wait(buf, s):
            pltpu.make_async_copy(g_hbm.at[src_v.at[0]], buf, s).wait()

        def _edges(i, _):
            c0 = _NBUF * i
            for k in range(_NBUF):
                _wait(bufs[k], sems[k])
                pltpu.sync_copy(bufs[k], acc_sh.at[dst_v.at[c0 + k]],
                                add=True)

                @pl.when(c0 + _NBUF + k < _WIN)
                def _():
                    _gather(c0 + _NBUF + k, bufs[k], sems[k])

            return 0

        def _window(w, _):
            pltpu.sync_copy(e3_hbm.at[0, wid, pl.ds(w * _WIN, _WIN)], src_v)
            pltpu.sync_copy(e3_hbm.at[1, wid, pl.ds(w * _WIN, _WIN)], dst_v)
            for k in range(_NBUF):
                _gather(k, bufs[k], sems[k])
            lax.fori_loop(0, _WIN // _NBUF, _edges, 0)
            return 0

        lax.fori_loop(0, rows_per_tile // _WIN, _window, 0)
        plsc.subcore_barrier()

        def _dump(out):
            sl = pl.ds(sid * n0, n0)
            pltpu.sync_copy(acc_sh.at[sl], out.at[sl])

            @pl.when(sid == _NS - 1)
            def _():
                tl = pl.ds(_NS * n0, tail)
                pltpu.sync_copy(acc_sh.at[tl], out.at[tl])

        @pl.when(cid == 0)
        def _():
            _dump(out0)

        @pl.when(cid == 1)
        def _():
            _dump(out1)

    return edge_kernel


def _matmul_tc(x_ref, w_ref, h_ref):
    h_ref[...] = jnp.dot(x_ref[...], w_ref[...],
                         preferred_element_type=jnp.float32)


def _dinv_scale_tc(p0_ref, p1_ref, h_ref, dinv_ref, g_ref):
    deg = 1.0 + p0_ref[:, 0:1] + p1_ref[:, 0:1]
    dinv = lax.rsqrt(deg)
    dinv_ref[...] = dinv
    g_ref[...] = h_ref[...] * dinv


def _combine_tc(g_ref, p0_ref, p1_ref, dinv_ref, b_ref, out_ref):
    s = g_ref[...] + p0_ref[...] + p1_ref[...]
    out_ref[...] = dinv_ref[...] * s + b_ref[...]


def kernel(x, edge_index, W, b):
    N, D_in = x.shape
    D_out = W.shape[1]
    E = edge_index.shape[1]

    e3 = edge_index.reshape(2, _NW, E // _CH // _NW, _CH)

    nb = 2000
    grid = (N // nb,)

    h = pl.pallas_call(
        _matmul_tc,
        grid=grid,
        in_specs=[
            pl.BlockSpec((nb, D_in), lambda i: (i, 0)),
            pl.BlockSpec((D_in, D_out), lambda i: (0, 0)),
        ],
        out_specs=pl.BlockSpec((nb, D_out), lambda i: (i, 0)),
        out_shape=jax.ShapeDtypeStruct((N, D_out), jnp.float32),
    )(x, W)

    hp0, hp1 = _make_deg_kernel(N, E)(e3)

    dinv, g = pl.pallas_call(
        _dinv_scale_tc,
        grid=grid,
        in_specs=[
            pl.BlockSpec((nb, 16), lambda i: (i, 0)),
            pl.BlockSpec((nb, 16), lambda i: (i, 0)),
            pl.BlockSpec((nb, D_out), lambda i: (i, 0)),
        ],
        out_specs=[
            pl.BlockSpec((nb, 1), lambda i: (i, 0)),
            pl.BlockSpec((nb, D_out), lambda i: (i, 0)),
        ],
        out_shape=[
            jax.ShapeDtypeStruct((N, 1), jnp.float32),
            jax.ShapeDtypeStruct((N, D_out), jnp.float32),
        ],
    )(hp0, hp1, h)

    p0, p1 = _make_edge_kernel(N, E, D_out)(e3, g)

    out = pl.pallas_call(
        _combine_tc,
        grid=grid,
        in_specs=[
            pl.BlockSpec((nb, D_out), lambda i: (i, 0)),
            pl.BlockSpec((nb, D_out), lambda i: (i, 0)),
            pl.BlockSpec((nb, D_out), lambda i: (i, 0)),
            pl.BlockSpec((nb, 1), lambda i: (i, 0)),
            pl.BlockSpec((1, D_out), lambda i: (0, 0)),
        ],
        out_specs=pl.BlockSpec((nb, D_out), lambda i: (i, 0)),
        out_shape=jax.ShapeDtypeStruct((N, D_out), jnp.float32),
    )(g, p0, p1, dinv, b.reshape(1, D_out))

    return out
